# baseline (speedup 1.0000x reference)
import jax
import jax.numpy as jnp
from jax import lax
from jax.experimental import pallas as pl
from jax.experimental.pallas import tpu as pltpu
from jax.experimental.pallas import tpu_sc as plsc

_NC = 2
_NS = 16
_NW = _NC * _NS
_LANES = 16

_B, _T, _D = 16, 2048, 512
_ROWS = _B * _T
_WPB = _NW // _B
_JPW = _D // _WPB
_IPW = (_T - _D) // _WPB
_CHUNK = 64
_NJ = _JPW // _CHUNK
_NBUF = 3
_IC = 128
_NI = _IPW // _IC


def _jitter_body(q_hbm, nb_hbm, out_hbm, nb_v, idx_v, buf_v, *sems):
    gsems, ssems, isems = sems[:_NBUF], sems[_NBUF:2 * _NBUF], sems[2 * _NBUF:]
    wid = lax.axis_index("s") * _NC + lax.axis_index("c")
    b = wid // _WPB
    half = wid % _WPB

    ibase = b * _T + _D + half * _IPW
    idmas = [
        pltpu.async_copy(
            q_hbm.at[pl.ds(ibase + i * _IC, _IC)],
            out_hbm.at[pl.ds(ibase + i * _IC, _IC)],
            isems[i],
        )
        for i in range(_NI)
    ]

    pltpu.sync_copy(nb_hbm, nb_v)
    iota = lax.iota(jnp.int32, _LANES)
    mlo = half * (_JPW // _LANES)
    mhi = mlo + _JPW // _LANES

    def resolve(m, carry):
        t0 = m * _LANES
        nbv = nb_v[pl.ds(t0, _LANES)]
        h = jnp.where(nbv == t0 + iota - 1, -1, nbv)
        v = jnp.maximum(plsc.cummax(h), carry)

        @pl.when(m >= mlo)
        def _():
            idx_v[pl.ds((m - mlo) * _LANES, _LANES)] = b * _T + v

        return jnp.max(v)

    lax.fori_loop(0, mhi, resolve, jnp.int32(-1))

    jbase = b * _T + half * _JPW

    def fetch(k):
        return pltpu.async_copy(
            q_hbm.at[idx_v.at[pl.ds(k * _CHUNK, _CHUNK)]],
            buf_v.at[k % _NBUF],
            gsems[k % _NBUF],
        )

    fetches = {k: fetch(k) for k in range(min(_NBUF, _NJ))}
    stores = {}
    for k in range(_NJ):
        fetches[k].wait()
        stores[k] = pltpu.async_copy(
            buf_v.at[k % _NBUF], out_hbm.at[pl.ds(jbase + k * _CHUNK, _CHUNK)],
            ssems[k % _NBUF],
        )
        if k + _NBUF < _NJ:
            stores[k].wait()
            fetches[k + _NBUF] = fetch(k + _NBUF)
    for k in range(max(0, _NJ - _NBUF), _NJ):
        stores[k].wait()

    for d in idmas:
        d.wait()


def kernel(quantized, neighbor_idx):
    q2d = quantized.reshape(_ROWS, _D)
    nb = jnp.asarray(neighbor_idx, jnp.int32)

    mesh = plsc.VectorSubcoreMesh(core_axis_name="c", subcore_axis_name="s")
    out = pl.kernel(
        _jitter_body,
        out_type=jax.ShapeDtypeStruct((_ROWS, _D), jnp.float32),
        mesh=mesh,
        scratch_types=[
            pltpu.VMEM((_D,), jnp.int32),
            pltpu.VMEM((_JPW,), jnp.int32),
            pltpu.VMEM((_NBUF, _CHUNK, _D), jnp.float32),
        ] + [pltpu.SemaphoreType.DMA] * (2 * _NBUF + _NI),
        compiler_params=pltpu.CompilerParams(needs_layout_passes=False),
    )(q2d, nb)
    return out.reshape(_B, _T, _D)


# speedup vs baseline: 23.3280x; 23.3280x over previous
import jax
import jax.numpy as jnp
from jax import lax
from jax.experimental import pallas as pl
from jax.experimental.pallas import tpu as pltpu
from jax.experimental.pallas import tpu_sc as plsc

_NC = 2
_NS = 16
_NW = _NC * _NS
_LANES = 16

_B, _T, _D = 16, 2048, 512
_ROWS = _B * _T
_WPB = _NW // _B
_JPW = _D // _WPB
_IPW = (_T - _D) // _WPB
_CHUNK = 64
_NJ = _JPW // _CHUNK
_NK = (_JPW + _IPW) // _CHUNK
_NBUF = 3


def _jitter_body(q_hbm, nb_hbm, out_hbm, nb_v, idx_v, buf_v, gsem, ssem):
    wid = lax.axis_index("s") * _NC + lax.axis_index("c")
    b = wid // _WPB
    half = wid % _WPB

    pltpu.sync_copy(nb_hbm, nb_v)

    iota = lax.iota(jnp.int32, _LANES)
    mlo = half * (_JPW // _LANES)
    mhi = mlo + _JPW // _LANES

    def resolve(m, carry):
        t0 = m * _LANES
        nbv = nb_v[pl.ds(t0, _LANES)]
        h = jnp.where(nbv == t0 + iota - 1, -1, nbv)
        v = jnp.maximum(plsc.cummax(h), carry)

        @pl.when(m >= mlo)
        def _():
            idx_v[pl.ds((m - mlo) * _LANES, _LANES)] = b * _T + v

        return jnp.max(v)

    lax.fori_loop(0, mhi, resolve, jnp.int32(-1))

    jbase = b * _T + half * _JPW
    ibase = b * _T + _D + half * _IPW

    def dst_base(k):
        return jbase + k * _CHUNK if k < _NJ else ibase + (k - _NJ) * _CHUNK

    def fetch(k):
        buf = buf_v.at[k % _NBUF]
        if k < _NJ:
            src = q_hbm.at[idx_v.at[pl.ds(k * _CHUNK, _CHUNK)]]
        else:
            src = q_hbm.at[pl.ds(dst_base(k), _CHUNK)]
        return pltpu.async_copy(src, buf, gsem)

    fetches = [fetch(k) for k in range(_NBUF)]
    stores = [None] * _NK
    for k in range(_NK):
        fetches[k % _NBUF].wait()
        stores[k] = pltpu.async_copy(
            buf_v.at[k % _NBUF], out_hbm.at[pl.ds(dst_base(k), _CHUNK)], ssem
        )
        if k + _NBUF < _NK:
            stores[k].wait()
            fetches[k % _NBUF] = fetch(k + _NBUF)
    for k in range(_NK - _NBUF, _NK):
        stores[k].wait()


def kernel(quantized, neighbor_idx):
    q2d = quantized.reshape(_ROWS, _D)
    nb = jnp.asarray(neighbor_idx, jnp.int32)

    mesh = plsc.VectorSubcoreMesh(core_axis_name="c", subcore_axis_name="s")
    out = pl.kernel(
        _jitter_body,
        out_type=jax.ShapeDtypeStruct((_ROWS, _D), jnp.float32),
        mesh=mesh,
        scratch_types=[
            pltpu.VMEM((_D,), jnp.int32),
            pltpu.VMEM((_JPW,), jnp.int32),
            pltpu.VMEM((_NBUF, _CHUNK, _D), jnp.float32),
            pltpu.SemaphoreType.DMA,
            pltpu.SemaphoreType.DMA,
        ],
        compiler_params=pltpu.CompilerParams(needs_layout_passes=False),
    )(q2d, nb)
    return out.reshape(_B, _T, _D)
